# baseline (speedup 1.0000x reference)
import functools

import jax
import jax.numpy as jnp
from jax import lax
from jax.experimental import pallas as pl
from jax.experimental.pallas import tpu as pltpu
from jax.experimental.pallas import tpu_sc as plsc

_NUM_VARS = 26
_CARD = 100000
_EMB = 64
_BATCH = 4096
_NC = 2
_NS = 16
_NW = _NC * _NS
_DPW = _EMB // _NW
_LANES = 16


def _build_kernel():
    mesh = plsc.VectorSubcoreMesh(core_axis_name="c", subcore_axis_name="s")
    out_type = tuple(
        jax.ShapeDtypeStruct((_EMB, _BATCH), jnp.float32)
        for _ in range(_NUM_VARS)
    )

    @functools.partial(
        pl.kernel,
        mesh=mesh,
        out_type=out_type,
        compiler_params=pltpu.CompilerParams(
            use_tc_tiling_on_sc=False, needs_layout_passes=False),
        scratch_types=[
            pltpu.VMEM_SHARED((_NS, _CARD), jnp.float32),
            pltpu.VMEM((_CARD,), jnp.float32),
            pltpu.VMEM((_BATCH,), jnp.int32),
            pltpu.VMEM((_BATCH,), jnp.float32),
            pltpu.VMEM((_BATCH,), jnp.float32),
            pltpu.SemaphoreType.DMA,
            pltpu.SemaphoreType.DMA,
            pltpu.SemaphoreType.DMA,
        ],
    )
    def k(idx_hbm, *rest):
        table_refs = rest[:_NUM_VARS]
        out_refs = rest[_NUM_VARS:2 * _NUM_VARS]
        shr_v, row_v, idx_v, out0_v, out1_v, rsem, isem, osem = (
            rest[2 * _NUM_VARS:])
        outs = (out0_v, out1_v)

        wid = lax.axis_index("s") * _NC + lax.axis_index("c")
        sid = lax.axis_index("s")
        d0 = wid * _DPW

        def gather_row(ov):
            if True:
                return
            @pl.loop(0, _BATCH, step=_LANES)
            def _(i):
                iv = idx_v[pl.ds(i, _LANES)]
                ov[pl.ds(i, _LANES)] = plsc.load_gather(row_v, [iv])

        _NSPLIT = 10
        _CHUNK = _CARD // _NSPLIT

        def load_row(t, d):
            return [
                pltpu.async_copy(
                    table_refs[t].at[d, pl.ds(q * _CHUNK, _CHUNK)],
                    shr_v.at[sid, pl.ds(q * _CHUNK, _CHUNK)], rsem)
                for q in range(_NSPLIT)
            ]

        out_copies = [None, None]
        icopy = pltpu.async_copy(idx_hbm.at[0, :], idx_v, isem)
        rcopy = load_row(0, d0)
        icopy.wait()
        for t in range(_NUM_VARS):
            for j in range(_DPW):
                for c in rcopy:
                    c.wait()
                ob = outs[j]
                gather_row(ob)
                if j + 1 < _DPW:
                    rcopy = load_row(t, d0 + j + 1)
                elif t + 1 < _NUM_VARS:
                    rcopy = load_row(t + 1, d0)
                if out_copies[j] is not None:
                    out_copies[j].wait()
                out_copies[j] = pltpu.async_copy(
                    ob, out_refs[t].at[d0 + j], osem)
            if t + 1 < _NUM_VARS:
                icopy = pltpu.async_copy(idx_hbm.at[t + 1, :], idx_v, isem)
                icopy.wait()
        out_copies[0].wait()
        out_copies[1].wait()

    return k


_sc_embed = _build_kernel()


def kernel(x, tables):
    xt = x.T
    tabts = tuple(t.T for t in tables)
    outs = _sc_embed(xt, *tabts)
    return tuple(o.T for o in outs)


# speedup vs baseline: 1.0331x; 1.0331x over previous
import functools

import jax
import jax.numpy as jnp
from jax import lax
from jax.experimental import pallas as pl
from jax.experimental.pallas import tpu as pltpu
from jax.experimental.pallas import tpu_sc as plsc

_NUM_VARS = 26
_CARD = 100000
_EMB = 64
_BATCH = 4096
_NC = 2
_NS = 16
_NW = _NC * _NS
_ROWS = _BATCH // _NW
_NBUF = 10
_LAG = 8


def _build_kernel():
    mesh = plsc.VectorSubcoreMesh(core_axis_name="c", subcore_axis_name="s")
    out_type = tuple(
        jax.ShapeDtypeStruct((_BATCH, _EMB), jnp.float32)
        for _ in range(_NUM_VARS)
    )

    @functools.partial(
        pl.kernel,
        mesh=mesh,
        out_type=out_type,
        compiler_params=pltpu.CompilerParams(use_tc_tiling_on_sc=False),
        scratch_types=(
            [pltpu.VMEM((_NUM_VARS, _ROWS), jnp.int32)]
            + [pltpu.VMEM((_ROWS, _EMB), jnp.float32) for _ in range(_NBUF)]
            + [pltpu.SemaphoreType.DMA for _ in range(2 * _NBUF)]
        ),
    )
    def k(idx_hbm, *rest):
        table_refs = rest[:_NUM_VARS]
        out_refs = rest[_NUM_VARS:2 * _NUM_VARS]
        scratch = rest[2 * _NUM_VARS:]
        idx_v = scratch[0]
        slabs = scratch[1:1 + _NBUF]
        gsems = scratch[1 + _NBUF:1 + 2 * _NBUF]
        osems = scratch[1 + 2 * _NBUF:1 + 3 * _NBUF]

        wid = lax.axis_index("s") * _NC + lax.axis_index("c")
        base = wid * _ROWS

        pltpu.sync_copy(idx_hbm.at[:, pl.ds(base, _ROWS)], idx_v)

        copies_g = [None] * _NUM_VARS
        copies_o = [None] * _NUM_VARS

        def drain_and_store(t):
            copies_g[t].wait()
            s = t % _NBUF
            copies_o[t] = pltpu.async_copy(
                slabs[s], out_refs[t].at[pl.ds(base, _ROWS)], osems[s])

        for t in range(_NUM_VARS):
            s = t % _NBUF
            if t >= _NBUF:
                copies_o[t - _NBUF].wait()
            copies_g[t] = pltpu.async_copy(
                table_refs[t].at[idx_v.at[t]], slabs[s], gsems[s])
            if t >= _LAG:
                drain_and_store(t - _LAG)
        for t in range(_NUM_VARS - _LAG, _NUM_VARS):
            drain_and_store(t)
        for t in range(_NUM_VARS - _NBUF, _NUM_VARS):
            copies_o[t].wait()

    return k


_sc_embed = _build_kernel()


def kernel(x, tables):
    xt = x.T
    return _sc_embed(xt, *tables)


# speedup vs baseline: 1.1922x; 1.1539x over previous
import functools

import jax
import jax.numpy as jnp
from jax import lax
from jax.experimental import pallas as pl
from jax.experimental.pallas import tpu as pltpu
from jax.experimental.pallas import tpu_sc as plsc

_NUM_VARS = 26
_CARD = 100000
_EMB = 64
_BATCH = 4096
_NC = 2
_NS = 16
_NW = _NC * _NS
_ROWS = _BATCH // _NW
_DPW = _EMB // _NW
_LANES = 16
_SPLIT = 13
_NREG = _NUM_VARS - _SPLIT
_RANGES = ((0, 33336), (33336, 33336), (66672, 33328))
_RBUF = 33336


def _build_kernel():
    mesh = plsc.VectorSubcoreMesh(core_axis_name="c", subcore_axis_name="s")
    out_type = tuple(
        jax.ShapeDtypeStruct((_EMB, _BATCH), jnp.float32)
        for _ in range(_NUM_VARS)
    )

    @functools.partial(
        pl.kernel,
        mesh=mesh,
        out_type=out_type,
        compiler_params=pltpu.CompilerParams(
            use_tc_tiling_on_sc=False, needs_layout_passes=False),
        scratch_types=[
            pltpu.VMEM((_SPLIT, _ROWS), jnp.int32),
            pltpu.VMEM((_EMB, _ROWS), jnp.float32),
            pltpu.VMEM((_EMB, _ROWS), jnp.float32),
            pltpu.VMEM((_BATCH,), jnp.int32),
            pltpu.VMEM((_RBUF,), jnp.float32),
            pltpu.VMEM((_RBUF,), jnp.float32),
            pltpu.VMEM((_BATCH,), jnp.float32),
            pltpu.VMEM((_BATCH,), jnp.float32),
            pltpu.SemaphoreType.DMA,
            pltpu.SemaphoreType.DMA,
            pltpu.SemaphoreType.DMA,
            pltpu.SemaphoreType.DMA,
            pltpu.SemaphoreType.DMA,
            pltpu.SemaphoreType.DMA,
            pltpu.SemaphoreType.DMA,
            pltpu.SemaphoreType.DMA,
        ],
    )
    def k(idx_hbm, *rest):
        table_refs = rest[:_NUM_VARS]
        out_refs = rest[_NUM_VARS:2 * _NUM_VARS]
        (sidx, slab0, slab1, ridx, rb0, rb1, ov0, ov1,
         gsem0, gsem1, sosem0, sosem1, rsem, rosem0, rosem1, risem
         ) = rest[2 * _NUM_VARS:]
        slabs = (slab0, slab1)
        gsems = (gsem0, gsem1)
        sosems = (sosem0, sosem1)
        rowbufs = (rb0, rb1)
        outvs = (ov0, ov1)
        rosems = (rosem0, rosem1)

        wid = lax.axis_index("s") * _NC + lax.axis_index("c")
        base = wid * _ROWS
        d0 = wid * _DPW

        pltpu.sync_copy(idx_hbm.at[pl.ds(0, _SPLIT), pl.ds(base, _ROWS)],
                        sidx)

        socopies = [None] * _SPLIT

        def fire_stream(i):
            s = i % 2

            @pl.loop(0, _EMB)
            def _(dd):
                pltpu.async_copy(
                    table_refs[i].at[dd].at[sidx.at[i]],
                    slabs[s].at[dd], gsems[s])

        def drain_stream(i):
            s = i % 2
            pltpu.make_async_copy(
                out_refs[i].at[:, pl.ds(0, _ROWS)], slabs[s], gsems[s]
            ).wait()
            socopies[i] = pltpu.async_copy(
                slabs[s], out_refs[i].at[:, pl.ds(base, _ROWS)], sosems[s])

        items = [(t, j, q)
                 for t in range(_SPLIT, _NUM_VARS)
                 for j in range(_DPW)
                 for q in range(len(_RANGES))]
        rcopies = [None] * len(items)
        rocopies = [None, None]

        def issue_item(kk):
            t, j, q = items[kk]
            lo, sz = _RANGES[q]
            rcopies[kk] = pltpu.async_copy(
                table_refs[t].at[d0 + j, pl.ds(lo, sz)],
                rowbufs[kk % 2].at[pl.ds(0, sz)], rsem)

        def merge(kk):
            t, j, q = items[kk]
            lo, sz = _RANGES[q]
            if q == 0 and rocopies[j] is not None:
                rocopies[j].wait()
            rcopies[kk].wait()
            ov = outvs[j]
            rb = rowbufs[kk % 2]

            @pl.loop(0, _BATCH, step=_LANES)
            def _(i):
                iv = ridx[pl.ds(i, _LANES)]
                ivc = jnp.minimum(jnp.maximum(iv - lo, 0), sz - 1)
                g = plsc.load_gather(rb, [ivc])
                valid = (iv >= lo) & (iv < lo + sz)
                ov[pl.ds(i, _LANES)] = jnp.where(valid, g,
                                                 ov[pl.ds(i, _LANES)])
            if q == len(_RANGES) - 1:
                rocopies[j] = pltpu.async_copy(
                    ov, out_refs[t].at[d0 + j], rosems[j])

        pltpu.async_copy(idx_hbm.at[_SPLIT], ridx, risem).wait()
        issue_item(0)
        issue_item(1)
        for i in range(_SPLIT):
            if i >= 2:
                socopies[i - 2].wait()
            fire_stream(i)
            for m in range(2 * len(_RANGES)):
                kk = i * 2 * len(_RANGES) + m
                merge(kk)
                if kk + 2 < len(items):
                    issue_item(kk + 2)
            if i + 1 < _NREG:
                pltpu.async_copy(
                    idx_hbm.at[_SPLIT + i + 1], ridx, risem).wait()
            drain_stream(i)
        socopies[_SPLIT - 2].wait()
        socopies[_SPLIT - 1].wait()
        rocopies[0].wait()
        rocopies[1].wait()

    return k


_sc_embed = _build_kernel()


def kernel(x, tables):
    xt = x.T
    tabts = tuple(t.T for t in tables)
    outs = _sc_embed(xt, *tabts)
    return tuple(o.T for o in outs)
